# baseline (speedup 1.0000x reference)
import functools

import jax
import jax.numpy as jnp
from jax import lax
from jax.experimental import pallas as pl
from jax.experimental.pallas import tpu as pltpu
from jax.experimental.pallas import tpu_sc as plsc

N = 10000
T = 8
D = 128
E = 320000

NC = 2
NS = 16
LB = 64
BLOCKS = 320
SEG = 16
NSEG = BLOCKS // SEG
BLOCKS_CAP = BLOCKS
E_PAD = NS * BLOCKS * LB
E_CAP = NS * BLOCKS_CAP * LB
N_ACC = 10496
RPT = N_ACC // NS
CHUNKS = T // NC

BN = 400
NB = N // BN


def _zero_rows(zbuf, acc, r0, nrows):
    zr = zbuf.shape[0]
    full, rem = nrows // zr, nrows % zr
    for z in range(full):
        pltpu.sync_copy(zbuf, acc.at[pl.ds(r0 + z * zr, zr)])
    if rem:
        pltpu.sync_copy(zbuf.at[pl.ds(0, rem)], acc.at[pl.ds(r0 + full * zr, rem)])


def _deg_body(dst_hbm, deg2_hbm, acc, onesb, zb, dstv):
    c = lax.axis_index("c")
    s = lax.axis_index("s")
    e0 = jnp.where(lax.iota(jnp.int32, 16) == 0, 1.0, 0.0).astype(jnp.float32)
    z16 = jnp.zeros((16,), jnp.float32)

    def init_row(r, _):
        for cc in range(8):
            onesb[r, pl.ds(cc * 16, 16)] = e0 if cc == 0 else z16
            zb[r % 32, pl.ds(cc * 16, 16)] = z16
        return 0

    lax.fori_loop(0, LB, init_row, 0)

    r0 = s * RPT
    _zero_rows(zb, acc, r0, RPT)
    plsc.subcore_barrier()

    def seg_loop(g, _):
        g0 = pl.multiple_of((g + c * (NSEG // 2)) * SEG, SEG)
        pltpu.sync_copy(dst_hbm.at[s, pl.ds(g0, SEG)], dstv)
        for j in range(SEG):
            pltpu.sync_copy(onesb, acc.at[dstv.at[j]], add=True)
        return 0

    lax.fori_loop(0, NSEG // 2, seg_loop, 0)
    plsc.subcore_barrier()
    for z in range(RPT // LB):
        pltpu.sync_copy(acc.at[pl.ds(r0 + z * LB, LB)], onesb)
        pltpu.sync_copy(onesb, deg2_hbm.at[c, pl.ds(r0 + z * LB, LB)])
    rem = RPT % LB
    if rem:
        zo = r0 + (RPT // LB) * LB
        pltpu.sync_copy(acc.at[pl.ds(zo, rem)], onesb.at[pl.ds(0, rem)])
        pltpu.sync_copy(onesb.at[pl.ds(0, rem)], deg2_hbm.at[c, pl.ds(zo, rem)])


_deg_kernel = functools.partial(
    pl.kernel,
    out_type=jax.ShapeDtypeStruct((NC, N_ACC, D), jnp.float32),
    mesh=plsc.VectorSubcoreMesh(
        core_axis_name="c", subcore_axis_name="s", num_cores=NC, num_subcores=NS
    ),
    scratch_types=[
        pltpu.VMEM_SHARED((N_ACC, D), jnp.float32),
        pltpu.VMEM((LB, D), jnp.float32),
        pltpu.VMEM((32, D), jnp.float32),
        pltpu.VMEM((SEG, LB), jnp.int32),
    ],
)(_deg_body)


def _pass_body(hwp, src_hbm, dst_hbm, s_out, acc, sb0, db0, rb0, rb1, rb2, rb3,
               zbuf, gsem0, gsem1, gsem2, gsem3):
    c = lax.axis_index("c")
    s = lax.axis_index("s")
    z16 = jnp.zeros((16,), jnp.float32)
    rbs = (rb0, rb1, rb2, rb3)
    gsems = (gsem0, gsem1, gsem2, gsem3)

    def zrow(r, _):
        for cc in range(8):
            zbuf[r, pl.ds(cc * 16, 16)] = z16
        return 0

    lax.fori_loop(0, zbuf.shape[0], zrow, 0)
    r0 = s * RPT

    for k in range(CHUNKS):
        t = c + NC * k
        _zero_rows(zbuf, acc, r0, RPT)
        plsc.subcore_barrier()

        def seg_loop(g, _):
            g0 = pl.multiple_of(g * SEG, SEG)
            pltpu.sync_copy(src_hbm.at[t, s, pl.ds(g0, SEG)], sb0)
            pltpu.sync_copy(dst_hbm.at[s, pl.ds(g0, SEG)], db0)
            pltpu.async_copy(hwp.at[sb0.at[0]], rbs[0], gsems[0])
            pltpu.async_copy(hwp.at[sb0.at[1]], rbs[1], gsems[1])
            for i in range(SEG // 2):
                for q in (2, 3):
                    jn = 2 * i + q
                    if jn < SEG:
                        pltpu.async_copy(hwp.at[sb0.at[jn]], rbs[jn % 4],
                                         gsems[jn % 4])
                for b in (0, 1):
                    j = 2 * i + b
                    pltpu.make_async_copy(hwp.at[sb0.at[j]], rbs[j % 4],
                                          gsems[j % 4]).wait()
                    pltpu.sync_copy(rbs[j % 4], acc.at[db0.at[j]], add=True)
            return 0

        lax.fori_loop(0, NSEG, seg_loop, 0)
        plsc.subcore_barrier()
        for z in range(RPT // LB):
            pltpu.sync_copy(acc.at[pl.ds(r0 + z * LB, LB)], rb0)
            pltpu.sync_copy(rb0, s_out.at[t, pl.ds(r0 + z * LB, LB)])
        rem = RPT % LB
        if rem:
            zo = r0 + (RPT // LB) * LB
            pltpu.sync_copy(acc.at[pl.ds(zo, rem)], rb0.at[pl.ds(0, rem)])
            pltpu.sync_copy(rb0.at[pl.ds(0, rem)], s_out.at[t, pl.ds(zo, rem)])


_pass_kernel = functools.partial(
    pl.kernel,
    out_type=jax.ShapeDtypeStruct((T, N_ACC, D), jnp.float32),
    mesh=plsc.VectorSubcoreMesh(
        core_axis_name="c", subcore_axis_name="s", num_cores=NC, num_subcores=NS
    ),
    scratch_types=[
        pltpu.VMEM_SHARED((N_ACC, D), jnp.float32),
        pltpu.VMEM((SEG, LB), jnp.int32),
        pltpu.VMEM((SEG, LB), jnp.int32),
        pltpu.VMEM((LB, D), jnp.float32),
        pltpu.VMEM((LB, D), jnp.float32),
        pltpu.VMEM((LB, D), jnp.float32),
        pltpu.VMEM((LB, D), jnp.float32),
        pltpu.VMEM((32, D), jnp.float32),
        pltpu.SemaphoreType.DMA,
        pltpu.SemaphoreType.DMA,
        pltpu.SemaphoreType.DMA,
        pltpu.SemaphoreType.DMA,
    ],
)(_pass_body)


def _dis(d_ref):
    deg = d_ref[0, :, 0:1] + d_ref[1, :, 0:1] + 1.0
    return lax.rsqrt(deg)


def _tc1_body(x_ref, w_ref, d_ref, o_ref):
    dis = _dis(d_ref)
    w = w_ref[...]
    for t in range(T):
        hw = jnp.dot(x_ref[:, t, :], w, preferred_element_type=jnp.float32)
        o_ref[t] = hw * dis


def _tc2_body(s_ref, h_ref, d_ref, b_ref, w_ref, o_ref):
    dis = _dis(d_ref)
    w = w_ref[...]
    b = b_ref[...]
    for t in range(T):
        h1 = jnp.maximum(dis * (s_ref[t] + h_ref[t]) + b, 0.0)
        o_ref[t] = jnp.dot(h1, w, preferred_element_type=jnp.float32) * dis


def _tc3_body(s_ref, h_ref, d_ref, b_ref, o_ref):
    dis = _dis(d_ref)
    b = b_ref[...]
    for t in range(T):
        o_ref[:, t, :] = jnp.maximum(dis * (s_ref[t] + h_ref[t]) + b, 0.0)


_deg_spec = pl.BlockSpec((NC, BN, D), lambda i: (0, i, 0))
_row_spec = pl.BlockSpec((T, BN, D), lambda i: (0, i, 0))
_x_spec = pl.BlockSpec((BN, T, D), lambda i: (i, 0, 0))
_w_spec = pl.BlockSpec((D, D), lambda i: (0, 0))
_b_spec = pl.BlockSpec((1, D), lambda i: (0, 0))

_tc1 = pl.pallas_call(
    _tc1_body,
    grid=(NB,),
    in_specs=[_x_spec, _w_spec, _deg_spec],
    out_specs=_row_spec,
    out_shape=jax.ShapeDtypeStruct((T, N, D), jnp.float32),
)

_tc2 = pl.pallas_call(
    _tc2_body,
    grid=(NB,),
    in_specs=[_row_spec, _row_spec, _deg_spec, _b_spec, _w_spec],
    out_specs=_row_spec,
    out_shape=jax.ShapeDtypeStruct((T, N, D), jnp.float32),
)

_tc3 = pl.pallas_call(
    _tc3_body,
    grid=(NB,),
    in_specs=[_row_spec, _row_spec, _deg_spec, _b_spec],
    out_specs=_x_spec,
    out_shape=jax.ShapeDtypeStruct((N, T, D), jnp.float32),
)


def kernel(x, edge_index, W1, b1, W2, b2):
    src = edge_index[0].astype(jnp.int32)
    dst = edge_index[1].astype(jnp.int32)
    srcp = jnp.zeros((NS, BLOCKS_CAP, LB), jnp.int32)
    srcp = srcp.at[:, :BLOCKS, :].set(
        jnp.concatenate([src, jnp.zeros((E_PAD - E,), jnp.int32)])
        .reshape(NS, BLOCKS, LB))
    dstp = jnp.full((NS, BLOCKS_CAP, LB), N, jnp.int32)
    dst_blk = dstp.at[:, :BLOCKS, :].set(
        jnp.concatenate([dst, jnp.full((E_PAD - E,), N, jnp.int32)])
        .reshape(NS, BLOCKS, LB))
    toff = (jnp.arange(T, dtype=jnp.int32) * N).reshape(T, 1, 1, 1)
    src_blk = srcp[None] + toff

    deg2 = _deg_kernel(dst_blk)
    hwp1 = _tc1(x, W1, deg2)
    s1 = _pass_kernel(hwp1.reshape(T * N, D), src_blk, dst_blk)
    hwp2 = _tc2(s1, hwp1, deg2, b1.reshape(1, D), W2)
    s2 = _pass_kernel(hwp2.reshape(T * N, D), src_blk, dst_blk)
    return _tc3(s2, hwp2, deg2, b2.reshape(1, D))


# speedup vs baseline: 1.0895x; 1.0895x over previous
import functools

import jax
import jax.numpy as jnp
from jax import lax
from jax.experimental import pallas as pl
from jax.experimental.pallas import tpu as pltpu
from jax.experimental.pallas import tpu_sc as plsc

N = 10000
T = 8
D = 128
E = 320000

NC = 2
NS = 16
LB = 128
BLOCKS = 160
SEG = 16
NSEG = BLOCKS // SEG
BLOCKS_CAP = BLOCKS
E_PAD = NS * BLOCKS * LB
E_CAP = NS * BLOCKS_CAP * LB
N_ACC = 10496
RPT = N_ACC // NS
CHUNKS = T // NC

BN = 400
NB = N // BN


def _zero_rows(zbuf, acc, r0, nrows):
    zr = zbuf.shape[0]
    full, rem = nrows // zr, nrows % zr
    for z in range(full):
        pltpu.sync_copy(zbuf, acc.at[pl.ds(r0 + z * zr, zr)])
    if rem:
        pltpu.sync_copy(zbuf.at[pl.ds(0, rem)], acc.at[pl.ds(r0 + full * zr, rem)])


def _deg_body(dst_hbm, deg2_hbm, acc, onesb, zb, dstv):
    c = lax.axis_index("c")
    s = lax.axis_index("s")
    e0 = jnp.where(lax.iota(jnp.int32, 16) == 0, 1.0, 0.0).astype(jnp.float32)
    z16 = jnp.zeros((16,), jnp.float32)

    def init_row(r, _):
        for cc in range(8):
            onesb[r, pl.ds(cc * 16, 16)] = e0 if cc == 0 else z16
            zb[r % 32, pl.ds(cc * 16, 16)] = z16
        return 0

    lax.fori_loop(0, LB, init_row, 0)

    r0 = s * RPT
    _zero_rows(zb, acc, r0, RPT)
    plsc.subcore_barrier()

    def seg_loop(g, _):
        g0 = pl.multiple_of((g + c * (NSEG // 2)) * SEG, SEG)
        pltpu.sync_copy(dst_hbm.at[s, pl.ds(g0, SEG)], dstv)
        for j in range(SEG):
            pltpu.sync_copy(onesb, acc.at[dstv.at[j]], add=True)
        return 0

    lax.fori_loop(0, NSEG // 2, seg_loop, 0)
    plsc.subcore_barrier()
    for z in range(RPT // LB):
        pltpu.sync_copy(acc.at[pl.ds(r0 + z * LB, LB)], onesb)
        pltpu.sync_copy(onesb, deg2_hbm.at[c, pl.ds(r0 + z * LB, LB)])
    rem = RPT % LB
    if rem:
        zo = r0 + (RPT // LB) * LB
        pltpu.sync_copy(acc.at[pl.ds(zo, rem)], onesb.at[pl.ds(0, rem)])
        pltpu.sync_copy(onesb.at[pl.ds(0, rem)], deg2_hbm.at[c, pl.ds(zo, rem)])


_deg_kernel = functools.partial(
    pl.kernel,
    out_type=jax.ShapeDtypeStruct((NC, N_ACC, D), jnp.float32),
    mesh=plsc.VectorSubcoreMesh(
        core_axis_name="c", subcore_axis_name="s", num_cores=NC, num_subcores=NS
    ),
    scratch_types=[
        pltpu.VMEM_SHARED((N_ACC, D), jnp.float32),
        pltpu.VMEM((LB, D), jnp.float32),
        pltpu.VMEM((32, D), jnp.float32),
        pltpu.VMEM((SEG, LB), jnp.int32),
    ],
)(_deg_body)


def _pass_body(hwp, src_hbm, dst_hbm, s_out, acc, sb0, db0, rb0, rb1, zbuf,
               gsem0, gsem1):
    c = lax.axis_index("c")
    s = lax.axis_index("s")
    z16 = jnp.zeros((16,), jnp.float32)
    rbs = (rb0, rb1)
    gsems = (gsem0, gsem1)

    def zrow(r, _):
        for cc in range(8):
            zbuf[r, pl.ds(cc * 16, 16)] = z16
        return 0

    lax.fori_loop(0, zbuf.shape[0], zrow, 0)
    r0 = s * RPT

    for k in range(CHUNKS):
        t = c + NC * k
        _zero_rows(zbuf, acc, r0, RPT)
        plsc.subcore_barrier()

        def seg_loop(g, _):
            g0 = pl.multiple_of(g * SEG, SEG)
            pltpu.sync_copy(src_hbm.at[t, s, pl.ds(g0, SEG)], sb0)
            pltpu.sync_copy(dst_hbm.at[s, pl.ds(g0, SEG)], db0)
            pltpu.async_copy(hwp.at[sb0.at[0]], rb0, gsem0)
            for j in range(SEG):
                if j + 1 < SEG:
                    pltpu.async_copy(hwp.at[sb0.at[j + 1]], rbs[(j + 1) % 2],
                                     gsems[(j + 1) % 2])
                pltpu.make_async_copy(hwp.at[sb0.at[j]], rbs[j % 2],
                                      gsems[j % 2]).wait()
                pltpu.sync_copy(rbs[j % 2], acc.at[db0.at[j]], add=True)
            return 0

        lax.fori_loop(0, NSEG, seg_loop, 0)
        plsc.subcore_barrier()
        for z in range(RPT // LB):
            pltpu.sync_copy(acc.at[pl.ds(r0 + z * LB, LB)], rb0)
            pltpu.sync_copy(rb0, s_out.at[t, pl.ds(r0 + z * LB, LB)])
        rem = RPT % LB
        if rem:
            zo = r0 + (RPT // LB) * LB
            pltpu.sync_copy(acc.at[pl.ds(zo, rem)], rb0.at[pl.ds(0, rem)])
            pltpu.sync_copy(rb0.at[pl.ds(0, rem)], s_out.at[t, pl.ds(zo, rem)])


_pass_kernel = functools.partial(
    pl.kernel,
    out_type=jax.ShapeDtypeStruct((T, N_ACC, D), jnp.float32),
    mesh=plsc.VectorSubcoreMesh(
        core_axis_name="c", subcore_axis_name="s", num_cores=NC, num_subcores=NS
    ),
    scratch_types=[
        pltpu.VMEM_SHARED((N_ACC, D), jnp.float32),
        pltpu.VMEM((SEG, LB), jnp.int32),
        pltpu.VMEM((SEG, LB), jnp.int32),
        pltpu.VMEM((LB, D), jnp.float32),
        pltpu.VMEM((LB, D), jnp.float32),
        pltpu.VMEM((32, D), jnp.float32),
        pltpu.SemaphoreType.DMA,
        pltpu.SemaphoreType.DMA,
    ],
)(_pass_body)


def _dis(d_ref):
    deg = d_ref[0, :, 0:1] + d_ref[1, :, 0:1] + 1.0
    return lax.rsqrt(deg)


def _tc1_body(x_ref, w_ref, d_ref, o_ref):
    dis = _dis(d_ref)
    w = w_ref[...]
    for t in range(T):
        hw = jnp.dot(x_ref[:, t, :], w, preferred_element_type=jnp.float32)
        o_ref[t] = hw * dis


def _tc2_body(s_ref, h_ref, d_ref, b_ref, w_ref, o_ref):
    dis = _dis(d_ref)
    w = w_ref[...]
    b = b_ref[...]
    for t in range(T):
        h1 = jnp.maximum(dis * (s_ref[t] + h_ref[t]) + b, 0.0)
        o_ref[t] = jnp.dot(h1, w, preferred_element_type=jnp.float32) * dis


def _tc3_body(s_ref, h_ref, d_ref, b_ref, o_ref):
    dis = _dis(d_ref)
    b = b_ref[...]
    for t in range(T):
        o_ref[:, t, :] = jnp.maximum(dis * (s_ref[t] + h_ref[t]) + b, 0.0)


_deg_spec = pl.BlockSpec((NC, BN, D), lambda i: (0, i, 0))
_row_spec = pl.BlockSpec((T, BN, D), lambda i: (0, i, 0))
_x_spec = pl.BlockSpec((BN, T, D), lambda i: (i, 0, 0))
_w_spec = pl.BlockSpec((D, D), lambda i: (0, 0))
_b_spec = pl.BlockSpec((1, D), lambda i: (0, 0))

_tc1 = pl.pallas_call(
    _tc1_body,
    grid=(NB,),
    in_specs=[_x_spec, _w_spec, _deg_spec],
    out_specs=_row_spec,
    out_shape=jax.ShapeDtypeStruct((T, N, D), jnp.float32),
)

_tc2 = pl.pallas_call(
    _tc2_body,
    grid=(NB,),
    in_specs=[_row_spec, _row_spec, _deg_spec, _b_spec, _w_spec],
    out_specs=_row_spec,
    out_shape=jax.ShapeDtypeStruct((T, N, D), jnp.float32),
)

_tc3 = pl.pallas_call(
    _tc3_body,
    grid=(NB,),
    in_specs=[_row_spec, _row_spec, _deg_spec, _b_spec],
    out_specs=_x_spec,
    out_shape=jax.ShapeDtypeStruct((N, T, D), jnp.float32),
)


def kernel(x, edge_index, W1, b1, W2, b2):
    src = edge_index[0].astype(jnp.int32)
    dst = edge_index[1].astype(jnp.int32)
    srcp = jnp.zeros((NS, BLOCKS_CAP, LB), jnp.int32)
    srcp = srcp.at[:, :BLOCKS, :].set(
        jnp.concatenate([src, jnp.zeros((E_PAD - E,), jnp.int32)])
        .reshape(NS, BLOCKS, LB))
    dstp = jnp.full((NS, BLOCKS_CAP, LB), N, jnp.int32)
    dst_blk = dstp.at[:, :BLOCKS, :].set(
        jnp.concatenate([dst, jnp.full((E_PAD - E,), N, jnp.int32)])
        .reshape(NS, BLOCKS, LB))
    toff = (jnp.arange(T, dtype=jnp.int32) * N).reshape(T, 1, 1, 1)
    src_blk = srcp[None] + toff

    deg2 = _deg_kernel(dst_blk)
    hwp1 = _tc1(x, W1, deg2)
    s1 = _pass_kernel(hwp1.reshape(T * N, D), src_blk, dst_blk)
    hwp2 = _tc2(s1, hwp1, deg2, b1.reshape(1, D), W2)
    s2 = _pass_kernel(hwp2.reshape(T * N, D), src_blk, dst_blk)
    return _tc3(s2, hwp2, deg2, b2.reshape(1, D))
